# baseline (speedup 1.0000x reference)
import jax
import jax.numpy as jnp
from jax import lax
from jax.experimental import pallas as pl
from jax.experimental.pallas import tpu as pltpu
from jax.experimental.pallas import tpu_sc as plsc

_B, _N, _C = 4, 8192, 2048
_S = 6144
_RT = 2048
_NT_CH = _S // _RT
_NSC = _N - _S
_NH = _NSC // 2
_RSC = 256
_NSC_CH = _NH // _RSC
_W = 128
_NG = _W // 16


def _tc_body(x_ref, oval_ref, oidx_ref, val_ref, idx_ref):
    c = pl.program_id(1)
    chunk = x_ref[0]
    lmin = jnp.min(chunk, axis=0)
    iota = lax.broadcasted_iota(jnp.int32, (_RT, _C), 0)
    masked = jnp.where(chunk == lmin[None, :], iota, _N)
    larg = jnp.min(masked, axis=0) + c * _RT

    @pl.when(c == 0)
    def _():
        val_ref[0] = lmin
        idx_ref[0] = larg

    @pl.when(c > 0)
    def _():
        better = lmin < val_ref[0]
        val_ref[0] = jnp.where(better, lmin, val_ref[0])
        idx_ref[0] = jnp.where(better, larg, idx_ref[0])

    @pl.when(c == _NT_CH - 1)
    def _():
        oval_ref[0, 0] = val_ref[0]
        oidx_ref[0, 0] = idx_ref[0]


def _tc_partial(x):
    return pl.pallas_call(
        _tc_body,
        grid=(_B, _NT_CH),
        in_specs=[pl.BlockSpec((1, _RT, _C), lambda b, c: (b, c, 0))],
        out_specs=(
            pl.BlockSpec((1, 1, _C), lambda b, c: (b, 0, 0)),
            pl.BlockSpec((1, 1, _C), lambda b, c: (b, 0, 0)),
        ),
        out_shape=(
            jax.ShapeDtypeStruct((_B, 1, _C), jnp.float32),
            jax.ShapeDtypeStruct((_B, 1, _C), jnp.int32),
        ),
        scratch_shapes=[
            pltpu.VMEM((1, _C), jnp.float32),
            pltpu.VMEM((1, _C), jnp.int32),
        ],
        compiler_params=pltpu.CompilerParams(
            dimension_semantics=("arbitrary", "arbitrary")
        ),
    )(x)


def _sc_partial(x):
    x2 = x.reshape(_B * _N, _C)

    @pl.kernel(
        out_type=(
            jax.ShapeDtypeStruct((2, _B, _C), jnp.float32),
            jax.ShapeDtypeStruct((2, _B, _C), jnp.int32),
        ),
        mesh=plsc.VectorSubcoreMesh(core_axis_name="c", subcore_axis_name="s"),
        scratch_types=[
            pltpu.VMEM((_RSC, _W), jnp.float32),
            pltpu.VMEM((_RSC, _W), jnp.float32),
            pltpu.VMEM((_W,), jnp.float32),
            pltpu.VMEM((_W,), jnp.int32),
            pltpu.SemaphoreType.DMA,
            pltpu.SemaphoreType.DMA,
            pltpu.SemaphoreType.DMA,
            pltpu.SemaphoreType.DMA,
        ],
    )
    def sc_kernel(x_hbm, oval_hbm, oidx_hbm, buf0, buf1, ov, oi,
                  sem0, sem1, osem0, osem1):
        core = lax.axis_index("c")
        sub = lax.axis_index("s")
        u = core * 16 + sub
        stripe = u % 16
        half = u // 16
        col0 = pl.multiple_of(stripe * _W, _W)
        rbase = pl.multiple_of(half * _NH, 8)

        bufs = (buf0, buf1)
        sems = (sem0, sem1)
        chunks = [(b, r) for b in range(_B) for r in range(_NSC_CH)]

        def dma_in(k):
            b, r = chunks[k]
            row0 = pl.multiple_of(b * _N + _S + r * _RSC + rbase, 8)
            return pltpu.make_async_copy(
                x_hbm.at[pl.ds(row0, _RSC), pl.ds(col0, _W)],
                bufs[k % 2],
                sems[k % 2],
            )

        dma_in(0).start()
        bests = None
        idxs = None
        half_i32 = jnp.full((16,), half * _NH, jnp.int32)
        for k, (b, r) in enumerate(chunks):
            if k + 1 < len(chunks):
                dma_in(k + 1).start()
            dma_in(k).wait()
            buf = bufs[k % 2]
            if r == 0:
                bests = [jnp.full((16,), jnp.inf, jnp.float32)
                         for _ in range(_NG)]
                idxs = [jnp.full((16,), 0, jnp.int32) for _ in range(_NG)]
            cur0 = jnp.full((16,), _S + r * _RSC, jnp.int32) + half_i32
            carry = (*bests, *idxs, cur0)

            def body(t, carry, buf=buf):
                bs = list(carry[0:_NG])
                ix = list(carry[_NG:2 * _NG])
                cur = carry[2 * _NG]
                for k2 in range(8):
                    row = t * 8 + k2
                    for g in range(_NG):
                        v = buf[row, pl.ds(g * 16, 16)]
                        cmp = v < bs[g]
                        bs[g] = jnp.minimum(v, bs[g])
                        ix[g] = jnp.where(cmp, cur, ix[g])
                    cur = cur + 1
                return (*bs, *ix, cur)

            carry = lax.fori_loop(0, _RSC // 8, body, carry)
            bests = list(carry[0:_NG])
            idxs = list(carry[_NG:2 * _NG])

            if r == _NSC_CH - 1:
                for g in range(_NG):
                    ov[pl.ds(g * 16, 16)] = bests[g]
                    oi[pl.ds(g * 16, 16)] = idxs[g]
                cpv = pltpu.make_async_copy(
                    ov, oval_hbm.at[half, b, pl.ds(col0, _W)], osem0)
                cpi = pltpu.make_async_copy(
                    oi, oidx_hbm.at[half, b, pl.ds(col0, _W)], osem1)
                cpv.start()
                cpi.start()
                cpv.wait()
                cpi.wait()

    return sc_kernel(x2)


def _merge_body(tcv_ref, tci_ref, scv_ref, sci_ref, o_ref):
    scv0, scv1 = scv_ref[0], scv_ref[1]
    sci0, sci1 = sci_ref[0], sci_ref[1]
    b1 = scv1 < scv0
    scv = jnp.where(b1, scv1, scv0)
    sci = jnp.where(b1, sci1, sci0)
    better = scv < tcv_ref[:, 0, :]
    o_ref[...] = jnp.where(better, sci, tci_ref[:, 0, :])


def _merge(tcv, tci, scv, sci):
    return pl.pallas_call(
        _merge_body,
        out_shape=jax.ShapeDtypeStruct((_B, _C), jnp.int32),
    )(tcv, tci, scv, sci)


def kernel(x):
    if _S == _N:
        _, tci = _tc_partial(x)
        return tci.reshape(_B, _C).astype(jnp.int64)
    tcv, tci = _tc_partial(x)
    scv, sci = _sc_partial(x)
    out = _merge(tcv, tci, scv, sci)
    return out.astype(jnp.int64)


# speedup vs baseline: 1.0085x; 1.0085x over previous
import jax
import jax.numpy as jnp
from jax import lax
from jax.experimental import pallas as pl
from jax.experimental.pallas import tpu as pltpu
from jax.experimental.pallas import tpu_sc as plsc

_B, _N, _C = 4, 8192, 2048
_S = 6144
_RT = 2048
_NT_CH = _S // _RT
_NSC = _N - _S
_NSLICE = 8
_RS = _NSC // _NSLICE
_CH = 16
_NCH = _RS // _CH
_G = 4
_NGB = _C // (16 * _G)


def _tc_body(x_ref, oval_ref, oidx_ref, val_ref, idx_ref):
    c = pl.program_id(1)
    chunk = x_ref[0]
    lmin = jnp.min(chunk, axis=0)
    iota = lax.broadcasted_iota(jnp.int32, (_RT, _C), 0)
    masked = jnp.where(chunk == lmin[None, :], iota, _N)
    larg = jnp.min(masked, axis=0) + c * _RT

    @pl.when(c == 0)
    def _():
        val_ref[0] = lmin
        idx_ref[0] = larg

    @pl.when(c > 0)
    def _():
        better = lmin < val_ref[0]
        val_ref[0] = jnp.where(better, lmin, val_ref[0])
        idx_ref[0] = jnp.where(better, larg, idx_ref[0])

    @pl.when(c == _NT_CH - 1)
    def _():
        oval_ref[0, 0] = val_ref[0]
        oidx_ref[0, 0] = idx_ref[0]


def _tc_partial(x):
    return pl.pallas_call(
        _tc_body,
        grid=(_B, _NT_CH),
        in_specs=[pl.BlockSpec((1, _RT, _C), lambda b, c: (b, c, 0))],
        out_specs=(
            pl.BlockSpec((1, 1, _C), lambda b, c: (b, 0, 0)),
            pl.BlockSpec((1, 1, _C), lambda b, c: (b, 0, 0)),
        ),
        out_shape=(
            jax.ShapeDtypeStruct((_B, 1, _C), jnp.float32),
            jax.ShapeDtypeStruct((_B, 1, _C), jnp.int32),
        ),
        scratch_shapes=[
            pltpu.VMEM((1, _C), jnp.float32),
            pltpu.VMEM((1, _C), jnp.int32),
        ],
        compiler_params=pltpu.CompilerParams(
            dimension_semantics=("arbitrary", "arbitrary")
        ),
    )(x)


def _sc_partial(x):
    x2 = x.reshape(_B * _N, _C)

    @pl.kernel(
        out_type=(
            jax.ShapeDtypeStruct((_NSLICE, _B, _C), jnp.float32),
            jax.ShapeDtypeStruct((_NSLICE, _B, _C), jnp.int32),
        ),
        mesh=plsc.VectorSubcoreMesh(core_axis_name="c", subcore_axis_name="s"),
        scratch_types=[
            pltpu.VMEM((_CH, _C), jnp.float32),
            pltpu.VMEM((_CH, _C), jnp.float32),
            pltpu.VMEM((_C,), jnp.float32),
            pltpu.VMEM((_C,), jnp.int32),
            pltpu.SemaphoreType.DMA,
            pltpu.SemaphoreType.DMA,
            pltpu.SemaphoreType.DMA,
            pltpu.SemaphoreType.DMA,
        ],
    )
    def sc_kernel(x_hbm, oval_hbm, oidx_hbm, buf0, buf1, bv, bi,
                  sem0, sem1, osem0, osem1):
        core = lax.axis_index("c")
        sub = lax.axis_index("s")
        u = core * 16 + sub
        b = u // _NSLICE
        sl = u % _NSLICE
        row_hbm0 = pl.multiple_of(b * _N + _S + sl * _RS, 8)
        loc0 = _S + sl * _RS

        bufs = (buf0, buf1)
        sems = (sem0, sem1)

        def dma_in(c):
            return pltpu.make_async_copy(
                x_hbm.at[pl.ds(row_hbm0 + c * _CH, _CH), :],
                bufs[c % 2],
                sems[c % 2],
            )

        inf16 = jnp.full((16,), jnp.inf, jnp.float32)
        zero16 = jnp.full((16,), 0, jnp.int32)

        @pl.loop(0, _C // 16)
        def _(g):
            bv[pl.ds(g * 16, 16)] = inf16
            bi[pl.ds(g * 16, 16)] = zero16

        dma_in(0).start()
        for c in range(_NCH):
            if c + 1 < _NCH:
                dma_in(c + 1).start()
            dma_in(c).wait()
            buf = bufs[c % 2]
            cur00 = loc0 + c * _CH

            def body(gb, _, buf=buf, cur00=cur00):
                col = gb * (16 * _G)
                bs = [bv[pl.ds(col + g * 16, 16)] for g in range(_G)]
                ix = [bi[pl.ds(col + g * 16, 16)] for g in range(_G)]
                cur = jnp.full((16,), 0, jnp.int32) + cur00
                for k2 in range(_CH):
                    for g in range(_G):
                        v = buf[k2, pl.ds(col + g * 16, 16)]
                        cmp = v < bs[g]
                        bs[g] = jnp.minimum(v, bs[g])
                        ix[g] = jnp.where(cmp, cur, ix[g])
                    cur = cur + 1
                for g in range(_G):
                    bv[pl.ds(col + g * 16, 16)] = bs[g]
                    bi[pl.ds(col + g * 16, 16)] = ix[g]
                return 0

            lax.fori_loop(0, _NGB, body, 0)

        cpv = pltpu.make_async_copy(bv, oval_hbm.at[sl, b, :], osem0)
        cpi = pltpu.make_async_copy(bi, oidx_hbm.at[sl, b, :], osem1)
        cpv.start()
        cpi.start()
        cpv.wait()
        cpi.wait()

    return sc_kernel(x2)


def _merge_body(tcv_ref, tci_ref, scv_ref, sci_ref, o_ref):
    val = tcv_ref[:, 0, :]
    idx = tci_ref[:, 0, :]
    for s in range(_NSLICE):
        v = scv_ref[s]
        i = sci_ref[s]
        better = v < val
        val = jnp.where(better, v, val)
        idx = jnp.where(better, i, idx)
    o_ref[...] = idx


def _merge(tcv, tci, scv, sci):
    return pl.pallas_call(
        _merge_body,
        out_shape=jax.ShapeDtypeStruct((_B, _C), jnp.int32),
    )(tcv, tci, scv, sci)


def kernel(x):
    if _S == _N:
        _, tci = _tc_partial(x)
        return tci.reshape(_B, _C).astype(jnp.int64)
    tcv, tci = _tc_partial(x)
    scv, sci = _sc_partial(x)
    out = _merge(tcv, tci, scv, sci)
    return out.astype(jnp.int64)


# speedup vs baseline: 1.1956x; 1.1856x over previous
import jax
import jax.numpy as jnp
from jax import lax
from jax.experimental import pallas as pl
from jax.experimental.pallas import tpu as pltpu
from jax.experimental.pallas import tpu_sc as plsc

_B, _N, _C = 4, 8192, 2048
_S = 8192
_RT = 2048
_NT_CH = _S // _RT
_NSC = _N - _S
_NSLICE = 8
_RS = _NSC // _NSLICE
_CH = 16
_NCH = _RS // _CH
_G = 4
_NGB = _C // (16 * _G)


def _tc_body(x_ref, oval_ref, oidx_ref, val_ref, idx_ref):
    c = pl.program_id(1)
    nblk = _RT // 8

    @pl.when(c == 0)
    def _():
        val_ref[...] = jnp.full((8, _C), jnp.inf, jnp.float32)
        idx_ref[...] = jnp.zeros((8, _C), jnp.int32)

    def body(r, carry):
        av, ai = carry
        blk = x_ref[0, pl.ds(pl.multiple_of(r * 8, 8), 8), :]
        cmp = blk < av
        av = jnp.minimum(blk, av)
        ai = jnp.where(cmp, jnp.full((8, _C), 0, jnp.int32) + (c * nblk + r),
                       ai)
        return av, ai

    acc = lax.fori_loop(0, nblk, body, (val_ref[...], idx_ref[...]))
    val_ref[...] = acc[0]
    idx_ref[...] = acc[1]

    @pl.when(c == _NT_CH - 1)
    def _():
        av = val_ref[...]
        row = idx_ref[...] * 8 + lax.broadcasted_iota(jnp.int32, (8, _C), 0)
        m = jnp.min(av, axis=0)
        cand = jnp.where(av == m[None, :], row, _N)
        oval_ref[0, 0] = m
        oidx_ref[0, 0] = jnp.min(cand, axis=0)


def _tc_partial(x):
    return pl.pallas_call(
        _tc_body,
        grid=(_B, _NT_CH),
        in_specs=[pl.BlockSpec((1, _RT, _C), lambda b, c: (b, c, 0))],
        out_specs=(
            pl.BlockSpec((1, 1, _C), lambda b, c: (b, 0, 0)),
            pl.BlockSpec((1, 1, _C), lambda b, c: (b, 0, 0)),
        ),
        out_shape=(
            jax.ShapeDtypeStruct((_B, 1, _C), jnp.float32),
            jax.ShapeDtypeStruct((_B, 1, _C), jnp.int32),
        ),
        scratch_shapes=[
            pltpu.VMEM((8, _C), jnp.float32),
            pltpu.VMEM((8, _C), jnp.int32),
        ],
        compiler_params=pltpu.CompilerParams(
            dimension_semantics=("arbitrary", "arbitrary")
        ),
    )(x)


def _sc_partial(x):
    x2 = x.reshape(_B * _N, _C)

    @pl.kernel(
        out_type=(
            jax.ShapeDtypeStruct((_NSLICE, _B, _C), jnp.float32),
            jax.ShapeDtypeStruct((_NSLICE, _B, _C), jnp.int32),
        ),
        mesh=plsc.VectorSubcoreMesh(core_axis_name="c", subcore_axis_name="s"),
        scratch_types=[
            pltpu.VMEM((_CH, _C), jnp.float32),
            pltpu.VMEM((_CH, _C), jnp.float32),
            pltpu.VMEM((_C,), jnp.float32),
            pltpu.VMEM((_C,), jnp.int32),
            pltpu.SemaphoreType.DMA,
            pltpu.SemaphoreType.DMA,
            pltpu.SemaphoreType.DMA,
            pltpu.SemaphoreType.DMA,
        ],
    )
    def sc_kernel(x_hbm, oval_hbm, oidx_hbm, buf0, buf1, bv, bi,
                  sem0, sem1, osem0, osem1):
        core = lax.axis_index("c")
        sub = lax.axis_index("s")
        u = core * 16 + sub
        b = u // _NSLICE
        sl = u % _NSLICE
        row_hbm0 = pl.multiple_of(b * _N + _S + sl * _RS, 8)
        loc0 = _S + sl * _RS

        bufs = (buf0, buf1)
        sems = (sem0, sem1)

        def dma_in(c):
            return pltpu.make_async_copy(
                x_hbm.at[pl.ds(row_hbm0 + c * _CH, _CH), :],
                bufs[c % 2],
                sems[c % 2],
            )

        inf16 = jnp.full((16,), jnp.inf, jnp.float32)
        zero16 = jnp.full((16,), 0, jnp.int32)

        @pl.loop(0, _C // 16)
        def _(g):
            bv[pl.ds(g * 16, 16)] = inf16
            bi[pl.ds(g * 16, 16)] = zero16

        dma_in(0).start()
        for c in range(_NCH):
            if c + 1 < _NCH:
                dma_in(c + 1).start()
            dma_in(c).wait()
            buf = bufs[c % 2]
            cur00 = loc0 + c * _CH

            def body(gb, _, buf=buf, cur00=cur00):
                col = gb * (16 * _G)
                bs = [bv[pl.ds(col + g * 16, 16)] for g in range(_G)]
                ix = [bi[pl.ds(col + g * 16, 16)] for g in range(_G)]
                cur = jnp.full((16,), 0, jnp.int32) + cur00
                for k2 in range(_CH):
                    for g in range(_G):
                        v = buf[k2, pl.ds(col + g * 16, 16)]
                        cmp = v < bs[g]
                        bs[g] = jnp.minimum(v, bs[g])
                        ix[g] = jnp.where(cmp, cur, ix[g])
                    cur = cur + 1
                for g in range(_G):
                    bv[pl.ds(col + g * 16, 16)] = bs[g]
                    bi[pl.ds(col + g * 16, 16)] = ix[g]
                return 0

            lax.fori_loop(0, _NGB, body, 0)

        cpv = pltpu.make_async_copy(bv, oval_hbm.at[sl, b, :], osem0)
        cpi = pltpu.make_async_copy(bi, oidx_hbm.at[sl, b, :], osem1)
        cpv.start()
        cpi.start()
        cpv.wait()
        cpi.wait()

    return sc_kernel(x2)


def _merge_body(tcv_ref, tci_ref, scv_ref, sci_ref, o_ref):
    val = tcv_ref[:, 0, :]
    idx = tci_ref[:, 0, :]
    for s in range(_NSLICE):
        v = scv_ref[s]
        i = sci_ref[s]
        better = v < val
        val = jnp.where(better, v, val)
        idx = jnp.where(better, i, idx)
    o_ref[...] = idx


def _merge(tcv, tci, scv, sci):
    return pl.pallas_call(
        _merge_body,
        out_shape=jax.ShapeDtypeStruct((_B, _C), jnp.int32),
    )(tcv, tci, scv, sci)


def kernel(x):
    if _S == _N:
        _, tci = _tc_partial(x)
        return tci.reshape(_B, _C).astype(jnp.int64)
    tcv, tci = _tc_partial(x)
    scv, sci = _sc_partial(x)
    out = _merge(tcv, tci, scv, sci)
    return out.astype(jnp.int64)


# speedup vs baseline: 1.2259x; 1.0253x over previous
import jax
import jax.numpy as jnp
from jax import lax
from jax.experimental import pallas as pl
from jax.experimental.pallas import tpu as pltpu
from jax.experimental.pallas import tpu_sc as plsc

_B, _N, _C = 4, 8192, 2048
_S = 8192
_RT = 1024
_NT_CH = _S // (2 * _RT)
_NSC = _N - _S
_NSLICE = 8
_RS = _NSC // _NSLICE
_CH = 16
_NCH = _RS // _CH
_G = 4
_NGB = _C // (16 * _G)


def _chunk_minarg(chunk, row0):
    lmin = jnp.min(chunk, axis=0)
    iota = lax.broadcasted_iota(jnp.int32, (_RT, _C), 0)
    masked = jnp.where(chunk == lmin[None, :], iota, _N)
    return lmin, jnp.min(masked, axis=0) + row0


def _tc_body(xa_ref, xb_ref, oval_ref, oidx_ref, val_ref, idx_ref):
    c = pl.program_id(1)
    amin, aarg = _chunk_minarg(xa_ref[0], (2 * c) * _RT)
    bmin, barg = _chunk_minarg(xb_ref[0], (2 * c + 1) * _RT)
    bb = bmin < amin
    lmin = jnp.where(bb, bmin, amin)
    larg = jnp.where(bb, barg, aarg)

    @pl.when(c == 0)
    def _():
        val_ref[0] = lmin
        idx_ref[0] = larg

    @pl.when(c > 0)
    def _():
        better = lmin < val_ref[0]
        val_ref[0] = jnp.where(better, lmin, val_ref[0])
        idx_ref[0] = jnp.where(better, larg, idx_ref[0])

    @pl.when(c == _NT_CH - 1)
    def _():
        oval_ref[0, 0] = val_ref[0]
        oidx_ref[0, 0] = idx_ref[0]


def _tc_partial(x):
    return pl.pallas_call(
        _tc_body,
        grid=(_B, _NT_CH),
        in_specs=[
            pl.BlockSpec((1, _RT, _C), lambda b, c: (b, 2 * c, 0)),
            pl.BlockSpec((1, _RT, _C), lambda b, c: (b, 2 * c + 1, 0)),
        ],
        out_specs=(
            pl.BlockSpec((1, 1, _C), lambda b, c: (b, 0, 0)),
            pl.BlockSpec((1, 1, _C), lambda b, c: (b, 0, 0)),
        ),
        out_shape=(
            jax.ShapeDtypeStruct((_B, 1, _C), jnp.float32),
            jax.ShapeDtypeStruct((_B, 1, _C), jnp.int32),
        ),
        scratch_shapes=[
            pltpu.VMEM((1, _C), jnp.float32),
            pltpu.VMEM((1, _C), jnp.int32),
        ],
        compiler_params=pltpu.CompilerParams(
            dimension_semantics=("arbitrary", "arbitrary")
        ),
    )(x, x)


def _sc_partial(x):
    x2 = x.reshape(_B * _N, _C)

    @pl.kernel(
        out_type=(
            jax.ShapeDtypeStruct((_NSLICE, _B, _C), jnp.float32),
            jax.ShapeDtypeStruct((_NSLICE, _B, _C), jnp.int32),
        ),
        mesh=plsc.VectorSubcoreMesh(core_axis_name="c", subcore_axis_name="s"),
        scratch_types=[
            pltpu.VMEM((_CH, _C), jnp.float32),
            pltpu.VMEM((_CH, _C), jnp.float32),
            pltpu.VMEM((_C,), jnp.float32),
            pltpu.VMEM((_C,), jnp.int32),
            pltpu.SemaphoreType.DMA,
            pltpu.SemaphoreType.DMA,
            pltpu.SemaphoreType.DMA,
            pltpu.SemaphoreType.DMA,
        ],
    )
    def sc_kernel(x_hbm, oval_hbm, oidx_hbm, buf0, buf1, bv, bi,
                  sem0, sem1, osem0, osem1):
        core = lax.axis_index("c")
        sub = lax.axis_index("s")
        u = core * 16 + sub
        b = u // _NSLICE
        sl = u % _NSLICE
        row_hbm0 = pl.multiple_of(b * _N + _S + sl * _RS, 8)
        loc0 = _S + sl * _RS

        bufs = (buf0, buf1)
        sems = (sem0, sem1)

        def dma_in(c):
            return pltpu.make_async_copy(
                x_hbm.at[pl.ds(row_hbm0 + c * _CH, _CH), :],
                bufs[c % 2],
                sems[c % 2],
            )

        inf16 = jnp.full((16,), jnp.inf, jnp.float32)
        zero16 = jnp.full((16,), 0, jnp.int32)

        @pl.loop(0, _C // 16)
        def _(g):
            bv[pl.ds(g * 16, 16)] = inf16
            bi[pl.ds(g * 16, 16)] = zero16

        dma_in(0).start()
        for c in range(_NCH):
            if c + 1 < _NCH:
                dma_in(c + 1).start()
            dma_in(c).wait()
            buf = bufs[c % 2]
            cur00 = loc0 + c * _CH

            def body(gb, _, buf=buf, cur00=cur00):
                col = gb * (16 * _G)
                bs = [bv[pl.ds(col + g * 16, 16)] for g in range(_G)]
                ix = [bi[pl.ds(col + g * 16, 16)] for g in range(_G)]
                cur = jnp.full((16,), 0, jnp.int32) + cur00
                for k2 in range(_CH):
                    for g in range(_G):
                        v = buf[k2, pl.ds(col + g * 16, 16)]
                        cmp = v < bs[g]
                        bs[g] = jnp.minimum(v, bs[g])
                        ix[g] = jnp.where(cmp, cur, ix[g])
                    cur = cur + 1
                for g in range(_G):
                    bv[pl.ds(col + g * 16, 16)] = bs[g]
                    bi[pl.ds(col + g * 16, 16)] = ix[g]
                return 0

            lax.fori_loop(0, _NGB, body, 0)

        cpv = pltpu.make_async_copy(bv, oval_hbm.at[sl, b, :], osem0)
        cpi = pltpu.make_async_copy(bi, oidx_hbm.at[sl, b, :], osem1)
        cpv.start()
        cpi.start()
        cpv.wait()
        cpi.wait()

    return sc_kernel(x2)


def _merge_body(tcv_ref, tci_ref, scv_ref, sci_ref, o_ref):
    val = tcv_ref[:, 0, :]
    idx = tci_ref[:, 0, :]
    for s in range(_NSLICE):
        v = scv_ref[s]
        i = sci_ref[s]
        better = v < val
        val = jnp.where(better, v, val)
        idx = jnp.where(better, i, idx)
    o_ref[...] = idx


def _merge(tcv, tci, scv, sci):
    return pl.pallas_call(
        _merge_body,
        out_shape=jax.ShapeDtypeStruct((_B, _C), jnp.int32),
    )(tcv, tci, scv, sci)


def kernel(x):
    if _S == _N:
        _, tci = _tc_partial(x)
        return tci.reshape(_B, _C).astype(jnp.int64)
    tcv, tci = _tc_partial(x)
    scv, sci = _sc_partial(x)
    out = _merge(tcv, tci, scv, sci)
    return out.astype(jnp.int64)
